# baseline (speedup 1.0000x reference)
import functools

import jax
import jax.numpy as jnp
from jax.experimental import pallas as pl
from jax.experimental.pallas import tpu as pltpu

STATE_DIM = 512
BUFFER_SIZE = 10000
MEM_SIZE = 100000
K_NOVELTY = 10
K_MEMORY = 32

GRID = 125
MEM_BLK = MEM_SIZE // GRID
BUF_BLK = BUFFER_SIZE // GRID


def _order_keys(x):
    b = jax.lax.bitcast_convert_type(x, jnp.int32)
    return b ^ jax.lax.shift_right_arithmetic(b, 31).__and__(jnp.int32(0x7FFFFFFF))


def _kth_largest(x, k):
    keys = _order_keys(x)

    def body(_, carry):
        lo, hi = carry
        mid = (jax.lax.shift_right_arithmetic(lo, 1)
               + jax.lax.shift_right_arithmetic(hi, 1)
               + ((lo | hi) & 1))
        mask = (keys >= mid).astype(jnp.int32)
        cnt = jnp.sum(mask, axis=0, keepdims=True).sum(axis=1, keepdims=True)
        big = cnt >= k
        return (jnp.where(big, mid, lo), jnp.where(big, hi, mid - 1))

    lo0 = jnp.full((1, 1), -(2**31), jnp.int32)
    hi0 = jnp.full((1, 1), 2**31 - 1, jnp.int32)
    lo, _ = jax.lax.fori_loop(0, 32, body, (lo0, hi0))
    inv = lo ^ jax.lax.shift_right_arithmetic(lo, 31).__and__(jnp.int32(0x7FFFFFFF))
    return jax.lax.bitcast_convert_type(inv, jnp.float32)


def _topk_sum(x, k):
    vk = _kth_largest(x, k)
    gt = x > vk
    s = jnp.sum(jnp.where(gt, x, 0.0), axis=0, keepdims=True).sum(
        axis=1, keepdims=True)
    c = jnp.sum(gt.astype(jnp.int32), axis=0, keepdims=True).sum(
        axis=1, keepdims=True)
    return s + (k - c).astype(jnp.float32) * vk


def _curiosity_kernel(state_hbm, mem_hbm, buf_hbm, out_ref,
                      scores_scr, dist_scr, state_scr,
                      mem_buf, buf_buf, state_sem, mem_sems, buf_sems):
    i = pl.program_id(0)
    slot = jax.lax.rem(i, 2)
    nslot = jax.lax.rem(i + 1, 2)

    def mem_copy(step, s_):
        return pltpu.make_async_copy(
            mem_hbm.at[pl.ds(step * MEM_BLK, MEM_BLK), :],
            mem_buf.at[s_], mem_sems.at[s_])

    def buf_copy(step, s_):
        return pltpu.make_async_copy(
            buf_hbm.at[pl.ds(step * BUF_BLK, BUF_BLK), :],
            buf_buf.at[s_], buf_sems.at[s_])

    @pl.when(i == 0)
    def _():
        pltpu.make_async_copy(state_hbm, state_scr, state_sem).start()
        mem_copy(0, 0).start()
        buf_copy(0, 0).start()
        pltpu.make_async_copy(state_hbm, state_scr, state_sem).wait()

    @pl.when(i + 1 < GRID)
    def _():
        mem_copy(i + 1, nslot).start()
        buf_copy(i + 1, nslot).start()

    mem_copy(i, slot).wait()
    buf_copy(i, slot).wait()

    s = state_scr[...]
    ones = jnp.ones((128, 1), jnp.float32)

    m = mem_buf[slot]
    pm = (m[:, 0:128] * s[:, 0:128] + m[:, 128:256] * s[:, 128:256]
          + m[:, 256:384] * s[:, 256:384] + m[:, 384:512] * s[:, 384:512])
    scores = jax.lax.dot_general(
        pm, ones,
        dimension_numbers=(((1,), (0,)), ((), ())),
        preferred_element_type=jnp.float32,
    )
    lane_s = jax.lax.broadcasted_iota(jnp.int32, (MEM_BLK, GRID), 1)
    scores_scr[...] = jnp.where(lane_s == i, scores, scores_scr[...])

    diff = buf_buf[slot] - s
    pd = (diff[:, 0:128] * diff[:, 0:128] + diff[:, 128:256] * diff[:, 128:256]
          + diff[:, 256:384] * diff[:, 256:384]
          + diff[:, 384:512] * diff[:, 384:512])
    d2 = jax.lax.dot_general(
        pd, ones,
        dimension_numbers=(((1,), (0,)), ((), ())),
        preferred_element_type=jnp.float32,
    )
    lane_d = jax.lax.broadcasted_iota(jnp.int32, (BUF_BLK, GRID), 1)
    dist_scr[...] = jnp.where(lane_d == i, jnp.sqrt(d2), dist_scr[...])

    @pl.when(i == GRID - 1)
    def _():
        mem_rel = _topk_sum(scores_scr[...], K_MEMORY) / K_MEMORY
        novelty = -_topk_sum(-dist_scr[...], K_NOVELTY) / K_NOVELTY
        out_ref[...] = novelty * mem_rel


@jax.jit
def kernel(state, action, state_buffer, memory_keys):
    del action
    state2d = state.reshape(1, STATE_DIM)
    out = pl.pallas_call(
        _curiosity_kernel,
        grid=(GRID,),
        in_specs=[
            pl.BlockSpec(memory_space=pl.ANY),
            pl.BlockSpec(memory_space=pl.ANY),
            pl.BlockSpec(memory_space=pl.ANY),
        ],
        out_specs=pl.BlockSpec((1, 1), lambda i: (0, 0)),
        out_shape=jax.ShapeDtypeStruct((1, 1), jnp.float32),
        scratch_shapes=[
            pltpu.VMEM((MEM_BLK, GRID), jnp.float32),
            pltpu.VMEM((BUF_BLK, GRID), jnp.float32),
            pltpu.VMEM((1, STATE_DIM), jnp.float32),
            pltpu.VMEM((2, MEM_BLK, STATE_DIM), jnp.float32),
            pltpu.VMEM((2, BUF_BLK, STATE_DIM), jnp.float32),
            pltpu.SemaphoreType.DMA,
            pltpu.SemaphoreType.DMA((2,)),
            pltpu.SemaphoreType.DMA((2,)),
        ],
    )(state2d, memory_keys, state_buffer)
    return out[0, 0]


# speedup vs baseline: 1.4391x; 1.4391x over previous
import functools

import jax
import jax.numpy as jnp
from jax.experimental import pallas as pl
from jax.experimental.pallas import tpu as pltpu

STATE_DIM = 512
BUFFER_SIZE = 10000
MEM_SIZE = 100000
K_NOVELTY = 10
K_MEMORY = 32

GRID = 50
MEM_BLK = MEM_SIZE // GRID
BUF_BLK = BUFFER_SIZE // GRID


def _order_keys(x):
    b = jax.lax.bitcast_convert_type(x, jnp.int32)
    return b ^ jax.lax.shift_right_arithmetic(b, 31).__and__(jnp.int32(0x7FFFFFFF))


def _kth_largest(x, k):
    keys = _order_keys(x)

    def body(_, carry):
        lo, hi = carry
        mid = (jax.lax.shift_right_arithmetic(lo, 1)
               + jax.lax.shift_right_arithmetic(hi, 1)
               + ((lo | hi) & 1))
        mask = (keys >= mid).astype(jnp.int32)
        cnt = jnp.sum(mask, axis=0, keepdims=True).sum(axis=1, keepdims=True)
        big = cnt >= k
        return (jnp.where(big, mid, lo), jnp.where(big, hi, mid - 1))

    lo0 = jnp.full((1, 1), -(2**31), jnp.int32)
    hi0 = jnp.full((1, 1), 2**31 - 1, jnp.int32)
    lo, _ = jax.lax.fori_loop(0, 32, body, (lo0, hi0))
    inv = lo ^ jax.lax.shift_right_arithmetic(lo, 31).__and__(jnp.int32(0x7FFFFFFF))
    return jax.lax.bitcast_convert_type(inv, jnp.float32)


def _topk_sum(x, k):
    vk = _kth_largest(x, k)
    gt = x > vk
    s = jnp.sum(jnp.where(gt, x, 0.0), axis=0, keepdims=True).sum(
        axis=1, keepdims=True)
    c = jnp.sum(gt.astype(jnp.int32), axis=0, keepdims=True).sum(
        axis=1, keepdims=True)
    return s + (k - c).astype(jnp.float32) * vk


def _curiosity_kernel(state_hbm, mem_hbm, buf_hbm, out_ref,
                      scores_scr, dist_scr, state_scr,
                      mem_buf, buf_buf, state_sem, mem_sems, buf_sems):
    i = pl.program_id(0)
    slot = jax.lax.rem(i, 2)
    nslot = jax.lax.rem(i + 1, 2)

    def mem_copy(step, s_):
        return pltpu.make_async_copy(
            mem_hbm.at[pl.ds(step * MEM_BLK, MEM_BLK), :],
            mem_buf.at[s_], mem_sems.at[s_])

    def buf_copy(step, s_):
        return pltpu.make_async_copy(
            buf_hbm.at[pl.ds(step * BUF_BLK, BUF_BLK), :],
            buf_buf.at[s_], buf_sems.at[s_])

    @pl.when(i == 0)
    def _():
        pltpu.make_async_copy(state_hbm, state_scr, state_sem).start()
        mem_copy(0, 0).start()
        buf_copy(0, 0).start()
        pltpu.make_async_copy(state_hbm, state_scr, state_sem).wait()

    @pl.when(i + 1 < GRID)
    def _():
        mem_copy(i + 1, nslot).start()
        buf_copy(i + 1, nslot).start()

    mem_copy(i, slot).wait()
    buf_copy(i, slot).wait()

    s = state_scr[...]
    ones = jnp.ones((128, 1), jnp.float32)

    m = mem_buf[slot]
    pm = (m[:, 0:128] * s[:, 0:128] + m[:, 128:256] * s[:, 128:256]
          + m[:, 256:384] * s[:, 256:384] + m[:, 384:512] * s[:, 384:512])
    scores = jax.lax.dot_general(
        pm, ones,
        dimension_numbers=(((1,), (0,)), ((), ())),
        preferred_element_type=jnp.float32,
    )
    lane_s = jax.lax.broadcasted_iota(jnp.int32, (MEM_BLK, GRID), 1)
    scores_scr[...] = jnp.where(lane_s == i, scores, scores_scr[...])

    diff = buf_buf[slot] - s
    pd = (diff[:, 0:128] * diff[:, 0:128] + diff[:, 128:256] * diff[:, 128:256]
          + diff[:, 256:384] * diff[:, 256:384]
          + diff[:, 384:512] * diff[:, 384:512])
    d2 = jax.lax.dot_general(
        pd, ones,
        dimension_numbers=(((1,), (0,)), ((), ())),
        preferred_element_type=jnp.float32,
    )
    lane_d = jax.lax.broadcasted_iota(jnp.int32, (BUF_BLK, GRID), 1)
    dist_scr[...] = jnp.where(lane_d == i, jnp.sqrt(d2), dist_scr[...])

    @pl.when(i == GRID - 1)
    def _():
        mem_rel = _topk_sum(scores_scr[...], K_MEMORY) / K_MEMORY
        novelty = -_topk_sum(-dist_scr[...], K_NOVELTY) / K_NOVELTY
        out_ref[...] = novelty * mem_rel


@jax.jit
def kernel(state, action, state_buffer, memory_keys):
    del action
    state2d = state.reshape(1, STATE_DIM)
    out = pl.pallas_call(
        _curiosity_kernel,
        grid=(GRID,),
        in_specs=[
            pl.BlockSpec(memory_space=pl.ANY),
            pl.BlockSpec(memory_space=pl.ANY),
            pl.BlockSpec(memory_space=pl.ANY),
        ],
        out_specs=pl.BlockSpec((1, 1), lambda i: (0, 0)),
        out_shape=jax.ShapeDtypeStruct((1, 1), jnp.float32),
        scratch_shapes=[
            pltpu.VMEM((MEM_BLK, GRID), jnp.float32),
            pltpu.VMEM((BUF_BLK, GRID), jnp.float32),
            pltpu.VMEM((1, STATE_DIM), jnp.float32),
            pltpu.VMEM((2, MEM_BLK, STATE_DIM), jnp.float32),
            pltpu.VMEM((2, BUF_BLK, STATE_DIM), jnp.float32),
            pltpu.SemaphoreType.DMA,
            pltpu.SemaphoreType.DMA((2,)),
            pltpu.SemaphoreType.DMA((2,)),
        ],
    )(state2d, memory_keys, state_buffer)
    return out[0, 0]
